# baseline (speedup 1.0000x reference)
import functools

import jax
import jax.numpy as jnp
from jax import lax
from jax.experimental import pallas as pl
from jax.experimental.pallas import tpu as pltpu
from jax.experimental.pallas import tpu_sc as plsc

N = 10000
E = 320000
DIN = 128
H = 64
G = 64
DOUT = 128

NC = 2
NS = 16
NW = NC * NS
K = 128
EPW = 10240
E_PAD = NW * EPW
C = EPW // K
N_PAD = 10240
RPS = N_PAD // NS
NB = 4
ZB = 32


def _dotT(a, w):
    return lax.dot_general(a, w, (((1,), (1,)), ((), ())),
                           preferred_element_type=jnp.float32)


def _tc_first_body(x_ref, wr_ref, wq_ref, b_ref, xw_ref, xr_ref):
    X = x_ref[...]
    xw_ref[pl.ds(0, N), :] = _dotT(X, wr_ref[...]).astype(jnp.bfloat16)
    xw_ref[pl.ds(N, N_PAD - N), :] = jnp.zeros((N_PAD - N, H), jnp.bfloat16)
    xr_ref[...] = _dotT(X, wq_ref[...]) + b_ref[...]


def _tc_first(x, w_rel, w_root, b):
    return pl.pallas_call(
        _tc_first_body,
        out_shape=(jax.ShapeDtypeStruct((N_PAD, H), jnp.bfloat16),
                   jax.ShapeDtypeStruct((N, H), jnp.float32)),
    )(x, w_rel, w_root, b)


def _tc_mid_body(p_ref, xr_ref, wr_ref, wq_ref, b_ref, xw_ref, xro_ref):
    agg = p_ref[0, :N].astype(jnp.float32) + p_ref[1, :N].astype(jnp.float32)
    h = jnp.maximum(agg + xr_ref[...], 0.0)
    xw_ref[pl.ds(0, N), :] = _dotT(h, wr_ref[...]).astype(jnp.bfloat16)
    xw_ref[pl.ds(N, N_PAD - N), :] = jnp.zeros((N_PAD - N, H), jnp.bfloat16)
    xro_ref[...] = _dotT(h, wq_ref[...]) + b_ref[...]


def _tc_mid(parts, xr, w_rel, w_root, b):
    return pl.pallas_call(
        _tc_mid_body,
        out_shape=(jax.ShapeDtypeStruct((N_PAD, H), jnp.bfloat16),
                   jax.ShapeDtypeStruct((N, H), jnp.float32)),
    )(parts, xr, w_rel, w_root, b)


def _tc_final_body(p_ref, xr_ref, bt_ref, wl_ref, bl_ref, out_ref):
    h = (p_ref[0, :N].astype(jnp.float32) + p_ref[1, :N].astype(jnp.float32)
         + xr_ref[...])
    gi = lax.broadcasted_iota(jnp.int32, (G, N), 0)
    A = (bt_ref[...] == gi).astype(jnp.float32)
    sums = jnp.dot(A, h, preferred_element_type=jnp.float32)
    counts = jnp.sum(A, axis=1, keepdims=True)
    pooled = sums / jnp.maximum(counts, 1.0)
    out_ref[...] = _dotT(pooled, wl_ref[...]) + bl_ref[...]


def _tc_final(parts, xr, batch2d, w_lin, b_lin):
    return pl.pallas_call(
        _tc_final_body,
        out_shape=jax.ShapeDtypeStruct((G, DOUT), jnp.float32),
    )(parts, xr, batch2d, w_lin, b_lin)


def _sc_seg_body(xw_hbm, src_hbm, dst_hbm, out_hbm,
                 src_v, dst_v, rows_v, zbuf, xw_sh, acc_sh, gsem, ssem):
    c = lax.axis_index("c")
    s = lax.axis_index("s")
    wid = s * NC + c

    pltpu.async_copy(xw_hbm.at[pl.ds(s * RPS, RPS)],
                     xw_sh.at[pl.ds(s * RPS, RPS)], gsem.at[0])
    zv = jnp.zeros((32,), jnp.bfloat16)

    def zrow(i, carry):
        for j in range(H // 32):
            zbuf[i, pl.ds(j * 32, 32)] = zv
        return carry
    lax.fori_loop(0, ZB, zrow, 0)

    def zcopy(t, carry):
        pltpu.async_copy(zbuf, acc_sh.at[pl.ds(s * RPS + t * ZB, ZB)],
                         ssem.at[0])
        return carry
    lax.fori_loop(0, RPS // ZB, zcopy, 0)

    r0 = wid * C
    pltpu.async_copy(src_hbm.at[pl.ds(r0, C)], src_v, gsem.at[1])
    pltpu.sync_copy(dst_hbm.at[pl.ds(r0, C)], dst_v)
    pltpu.make_async_copy(src_hbm.at[pl.ds(r0, C)], src_v, gsem.at[1]).wait()
    pltpu.make_async_copy(xw_hbm.at[pl.ds(s * RPS, RPS)],
                          xw_sh.at[pl.ds(s * RPS, RPS)], gsem.at[0]).wait()

    def zdrain(t, carry):
        pltpu.make_async_copy(zbuf, acc_sh.at[pl.ds(s * RPS + t * ZB, ZB)],
                              ssem.at[0]).wait()
        return carry
    lax.fori_loop(0, RPS // ZB, zdrain, 0)
    plsc.subcore_barrier()

    for b in range(NB):
        pltpu.async_copy(xw_sh.at[src_v.at[b]], rows_v.at[b], gsem.at[b])

    @pl.loop(0, C, step=NB)
    def _chunks(i):
        for b in range(NB):
            ch = i + b
            pltpu.make_async_copy(xw_sh.at[src_v.at[ch]], rows_v.at[b],
                                  gsem.at[b]).wait()
            pltpu.async_copy(rows_v.at[b], acc_sh.at[dst_v.at[ch]],
                             ssem.at[b], add=True)
        for b in range(NB):
            ch = i + b
            pltpu.make_async_copy(rows_v.at[b], acc_sh.at[dst_v.at[ch]],
                                  ssem.at[b]).wait()

            @pl.when(i + NB < C)
            def _():
                pltpu.async_copy(xw_sh.at[src_v.at[ch + NB]], rows_v.at[b],
                                 gsem.at[b])

    plsc.subcore_barrier()
    pltpu.sync_copy(acc_sh.at[pl.ds(s * RPS, RPS)],
                    out_hbm.at[c, pl.ds(s * RPS, RPS)])


@functools.cache
def _sc_seg_kernel():
    return pl.kernel(
        _sc_seg_body,
        out_type=jax.ShapeDtypeStruct((NC, N_PAD, H), jnp.bfloat16),
        mesh=plsc.VectorSubcoreMesh(core_axis_name="c", subcore_axis_name="s",
                                    num_cores=NC, num_subcores=NS),
        scratch_types=[
            pltpu.VMEM((C, K), jnp.int32),
            pltpu.VMEM((C, K), jnp.int32),
            pltpu.VMEM((NB, K, H), jnp.bfloat16),
            pltpu.VMEM((ZB, H), jnp.bfloat16),
            pltpu.VMEM_SHARED((N_PAD, H), jnp.bfloat16),
            pltpu.VMEM_SHARED((N_PAD, H), jnp.bfloat16),
            pltpu.SemaphoreType.DMA((NB,)),
            pltpu.SemaphoreType.DMA((NB,)),
        ],
        compiler_params=pltpu.CompilerParams(use_tc_tiling_on_sc=False),
    )


def _sc_seg(xw, src2d, dst2d):
    return _sc_seg_kernel()(xw, src2d, dst2d)


def kernel(x, edge_index, edge_attr, batch,
           W1_rel, b1, W1_root, W2_rel, b2, W2_root,
           W3_rel, b3, W3_root, W_lin, b_lin):
    del edge_attr
    npad = E_PAD - E
    src = jnp.concatenate([edge_index[0], jnp.zeros((npad,), jnp.int32)])
    dst = jnp.concatenate([edge_index[1],
                           N + (jnp.arange(npad, dtype=jnp.int32) % (N_PAD - N))])
    src2d = src.reshape(E_PAD // K, K)
    dst2d = dst.reshape(E_PAD // K, K)
    batch2d = batch.reshape(1, N)
    b1r = b1.reshape(1, H)
    b2r = b2.reshape(1, H)
    b3r = b3.reshape(1, H)
    blr = b_lin.reshape(1, DOUT)

    xw1, xr1 = _tc_first(x, W1_rel, W1_root, b1r)
    p1 = _sc_seg(xw1, src2d, dst2d)
    xw2, xr2 = _tc_mid(p1, xr1, W2_rel, W2_root, b2r)
    p2 = _sc_seg(xw2, src2d, dst2d)
    xw3, xr3 = _tc_mid(p2, xr2, W3_rel, W3_root, b3r)
    p3 = _sc_seg(xw3, src2d, dst2d)
    return _tc_final(p3, xr3, batch2d, W_lin, b_lin)


# speedup vs baseline: 1.0627x; 1.0627x over previous
import functools

import jax
import jax.numpy as jnp
from jax import lax
from jax.experimental import pallas as pl
from jax.experimental.pallas import tpu as pltpu
from jax.experimental.pallas import tpu_sc as plsc

N = 10000
E = 320000
DIN = 128
H = 64
G = 64
DOUT = 128

NC = 2
NS = 16
NW = NC * NS
K = 128
EPW = 10240
E_PAD = NW * EPW
C = EPW // K
N_PAD = 10240
RPS = N_PAD // NS
NB = 2
ZB = 32


def _dotT(a, w):
    return lax.dot_general(a, w, (((1,), (1,)), ((), ())),
                           preferred_element_type=jnp.float32)


def _tc_first_body(x_ref, wr_ref, wq_ref, b_ref, xw_ref, xr_ref):
    X = x_ref[...]
    xw_ref[pl.ds(0, N), :] = _dotT(X, wr_ref[...]).astype(jnp.bfloat16)
    xw_ref[pl.ds(N, N_PAD - N), :] = jnp.zeros((N_PAD - N, H), jnp.bfloat16)
    xr_ref[...] = _dotT(X, wq_ref[...]) + b_ref[...]


def _tc_first(x, w_rel, w_root, b):
    return pl.pallas_call(
        _tc_first_body,
        out_shape=(jax.ShapeDtypeStruct((N_PAD, H), jnp.bfloat16),
                   jax.ShapeDtypeStruct((N, H), jnp.float32)),
    )(x, w_rel, w_root, b)


def _tc_mid_body(p_ref, xr_ref, wr_ref, wq_ref, b_ref, xw_ref, xro_ref):
    agg = p_ref[0, :N].astype(jnp.float32) + p_ref[1, :N].astype(jnp.float32)
    h = jnp.maximum(agg + xr_ref[...], 0.0)
    xw_ref[pl.ds(0, N), :] = _dotT(h, wr_ref[...]).astype(jnp.bfloat16)
    xw_ref[pl.ds(N, N_PAD - N), :] = jnp.zeros((N_PAD - N, H), jnp.bfloat16)
    xro_ref[...] = _dotT(h, wq_ref[...]) + b_ref[...]


def _tc_mid(parts, xr, w_rel, w_root, b):
    return pl.pallas_call(
        _tc_mid_body,
        out_shape=(jax.ShapeDtypeStruct((N_PAD, H), jnp.bfloat16),
                   jax.ShapeDtypeStruct((N, H), jnp.float32)),
    )(parts, xr, w_rel, w_root, b)


def _tc_final_body(p_ref, xr_ref, bt_ref, wl_ref, bl_ref, out_ref):
    h = (p_ref[0, :N].astype(jnp.float32) + p_ref[1, :N].astype(jnp.float32)
         + xr_ref[...])
    gi = lax.broadcasted_iota(jnp.int32, (G, N), 0)
    A = (bt_ref[...] == gi).astype(jnp.float32)
    sums = jnp.dot(A, h, preferred_element_type=jnp.float32)
    counts = jnp.sum(A, axis=1, keepdims=True)
    pooled = sums / jnp.maximum(counts, 1.0)
    out_ref[...] = _dotT(pooled, wl_ref[...]) + bl_ref[...]


def _tc_final(parts, xr, batch2d, w_lin, b_lin):
    return pl.pallas_call(
        _tc_final_body,
        out_shape=jax.ShapeDtypeStruct((G, DOUT), jnp.float32),
    )(parts, xr, batch2d, w_lin, b_lin)


def _sc_seg_body(xw_hbm, src_hbm, dst_hbm, out_hbm,
                 src_v, dst_v, rows_v, zbuf, xw_sh, acc_sh, gsem, ssem):
    c = lax.axis_index("c")
    s = lax.axis_index("s")
    wid = s * NC + c

    pltpu.async_copy(xw_hbm.at[pl.ds(s * RPS, RPS)],
                     xw_sh.at[pl.ds(s * RPS, RPS)], gsem.at[0])
    zv = jnp.zeros((32,), jnp.bfloat16)

    def zrow(i, carry):
        for j in range(H // 32):
            zbuf[i, pl.ds(j * 32, 32)] = zv
        return carry
    lax.fori_loop(0, ZB, zrow, 0)

    def zcopy(t, carry):
        pltpu.async_copy(zbuf, acc_sh.at[pl.ds(s * RPS + t * ZB, ZB)],
                         ssem.at[0])
        return carry
    lax.fori_loop(0, RPS // ZB, zcopy, 0)

    r0 = wid * C
    pltpu.async_copy(src_hbm.at[pl.ds(r0, C)], src_v, gsem.at[1])
    pltpu.sync_copy(dst_hbm.at[pl.ds(r0, C)], dst_v)
    pltpu.make_async_copy(src_hbm.at[pl.ds(r0, C)], src_v, gsem.at[1]).wait()
    pltpu.make_async_copy(xw_hbm.at[pl.ds(s * RPS, RPS)],
                          xw_sh.at[pl.ds(s * RPS, RPS)], gsem.at[0]).wait()

    def zdrain(t, carry):
        pltpu.make_async_copy(zbuf, acc_sh.at[pl.ds(s * RPS + t * ZB, ZB)],
                              ssem.at[0]).wait()
        return carry
    lax.fori_loop(0, RPS // ZB, zdrain, 0)
    plsc.subcore_barrier()

    for b in range(NB):
        pltpu.async_copy(xw_sh.at[src_v.at[b]], rows_v.at[b], gsem.at[b])

    @pl.loop(0, C, step=NB)
    def _chunks(i):
        for b in range(NB):
            ch = i + b
            pltpu.make_async_copy(xw_sh.at[src_v.at[ch]], rows_v.at[b],
                                  gsem.at[b]).wait()
            pltpu.async_copy(rows_v.at[b], acc_sh.at[dst_v.at[ch]],
                             ssem.at[b], add=True)
        for b in range(NB):
            ch = i + b
            pltpu.make_async_copy(rows_v.at[b], acc_sh.at[dst_v.at[ch]],
                                  ssem.at[b]).wait()

            @pl.when(i + NB < C)
            def _():
                pltpu.async_copy(xw_sh.at[src_v.at[ch + NB]], rows_v.at[b],
                                 gsem.at[b])

    plsc.subcore_barrier()
    pltpu.sync_copy(acc_sh.at[pl.ds(s * RPS, RPS)],
                    out_hbm.at[c, pl.ds(s * RPS, RPS)])


@functools.cache
def _sc_seg_kernel():
    return pl.kernel(
        _sc_seg_body,
        out_type=jax.ShapeDtypeStruct((NC, N_PAD, H), jnp.bfloat16),
        mesh=plsc.VectorSubcoreMesh(core_axis_name="c", subcore_axis_name="s",
                                    num_cores=NC, num_subcores=NS),
        scratch_types=[
            pltpu.VMEM((C, K), jnp.int32),
            pltpu.VMEM((C, K), jnp.int32),
            pltpu.VMEM((NB, K, H), jnp.bfloat16),
            pltpu.VMEM((ZB, H), jnp.bfloat16),
            pltpu.VMEM_SHARED((N_PAD, H), jnp.bfloat16),
            pltpu.VMEM_SHARED((N_PAD, H), jnp.bfloat16),
            pltpu.SemaphoreType.DMA((NB,)),
            pltpu.SemaphoreType.DMA((NB,)),
        ],
        compiler_params=pltpu.CompilerParams(use_tc_tiling_on_sc=False),
    )


def _sc_seg(xw, src2d, dst2d):
    return _sc_seg_kernel()(xw, src2d, dst2d)


def kernel(x, edge_index, edge_attr, batch,
           W1_rel, b1, W1_root, W2_rel, b2, W2_root,
           W3_rel, b3, W3_root, W_lin, b_lin):
    del edge_attr
    npad = E_PAD - E
    src = jnp.concatenate([edge_index[0], jnp.zeros((npad,), jnp.int32)])
    dst = jnp.concatenate([edge_index[1],
                           N + (jnp.arange(npad, dtype=jnp.int32) % (N_PAD - N))])
    src2d = src.reshape(E_PAD // K, K)
    dst2d = dst.reshape(E_PAD // K, K)
    batch2d = batch.reshape(1, N)
    b1r = b1.reshape(1, H)
    b2r = b2.reshape(1, H)
    b3r = b3.reshape(1, H)
    blr = b_lin.reshape(1, DOUT)

    xw1, xr1 = _tc_first(x, W1_rel, W1_root, b1r)
    p1 = _sc_seg(xw1, src2d, dst2d)
    xw2, xr2 = _tc_mid(p1, xr1, W2_rel, W2_root, b2r)
    p2 = _sc_seg(xw2, src2d, dst2d)
    xw3, xr3 = _tc_mid(p2, xr2, W3_rel, W3_root, b3r)
    p3 = _sc_seg(xw3, src2d, dst2d)
    return _tc_final(p3, xr3, batch2d, W_lin, b_lin)
